# Initial kernel scaffold; baseline (speedup 1.0000x reference)
import jax
import jax.numpy as jnp
from jax.experimental import pallas as pl


def kernel(x_t, t, semantic_tokens, speaker_ids, sem_emb, te_w1, te_b1, te_w2, te_b2, spk_emb, spk_w, spk_b, cond_w, cond_b, in_w, in_b, n1_w, n1_b, qkv_w, attn_out_w, n2_w, n2_b, router_w, e_w1, e_b1, e_w2, e_b2, on_g, on_b, op_w, op_b):
    raise NotImplementedError("write your pallas kernel here")


# speedup vs baseline: 2.0138x; 2.0138x over previous
import functools

import jax
import jax.numpy as jnp
from jax import lax
from jax.experimental import pallas as pl
from jax.experimental.pallas import tpu as pltpu
from jax.experimental.pallas import tpu_sc as plsc

B, T, A = 2, 2048, 80
D, C, H = 768, 256, 12
E, TOPK, FF = 8, 2, 3072
HD = D // H
N = B * T
EPS = 1e-05

_SC_NC = 2
_SC_NS = 16
_SC_NW = _SC_NC * _SC_NS


def _sem_gather_sc(sem_emb, tokens):
    b_per_w = N // _SC_NW
    mesh = plsc.VectorSubcoreMesh(core_axis_name="c", subcore_axis_name="s")

    @functools.partial(
        pl.kernel,
        mesh=mesh,
        out_type=jax.ShapeDtypeStruct((N, C), jnp.float32),
        scratch_types=[
            pltpu.VMEM((b_per_w,), jnp.int32),
            pltpu.VMEM((b_per_w, C), jnp.float32),
            pltpu.SemaphoreType.DMA,
        ],
    )
    def k(table_hbm, idx_hbm, out_hbm, idx_v, rows_v, sem):
        wid = lax.axis_index("s") * _SC_NC + lax.axis_index("c")
        base = wid * b_per_w
        pltpu.sync_copy(idx_hbm.at[pl.ds(base, b_per_w)], idx_v)
        pltpu.async_copy(table_hbm.at[idx_v], rows_v, sem).wait()
        pltpu.sync_copy(rows_v, out_hbm.at[pl.ds(base, b_per_w)])

    return k(sem_emb, tokens)


def _prelude_kernel(t_ref, ids_ref, spk_emb_ref, tw1_ref, tb1_ref, tw2_ref,
                    tb2_ref, sw_ref, sb_ref, ts_ref):
    half = C // 2
    i = lax.broadcasted_iota(jnp.int32, (1, half), 1).astype(jnp.float32)
    freqs = jnp.exp(-jnp.log(10000.0) * i / half)
    ang = t_ref[...] * freqs
    emb = jnp.concatenate([jnp.sin(ang), jnp.cos(ang)], axis=-1)
    h = jnp.dot(emb, tw1_ref[...], preferred_element_type=jnp.float32) + tb1_ref[...]
    h = h * jax.nn.sigmoid(h)
    tc = jnp.dot(h, tw2_ref[...], preferred_element_type=jnp.float32) + tb2_ref[...]
    rows = [spk_emb_ref[pl.ds(ids_ref[b], 1), :] for b in range(B)]
    spk_rows = jnp.concatenate(rows, axis=0)
    spk = jnp.dot(spk_rows, sw_ref[...], preferred_element_type=jnp.float32) + sb_ref[...]
    ts_ref[...] = jnp.concatenate([tc, spk], axis=-1)


def _prelude(t, speaker_ids, spk_emb, te_w1, te_b1, te_w2, te_b2, spk_w,
             spk_b):
    return pl.pallas_call(
        _prelude_kernel,
        out_shape=jax.ShapeDtypeStruct((B, 2 * C), jnp.float32),
        in_specs=[
            pl.BlockSpec(memory_space=pltpu.VMEM),
            pl.BlockSpec(memory_space=pltpu.SMEM),
            pl.BlockSpec(memory_space=pltpu.VMEM),
            pl.BlockSpec(memory_space=pltpu.VMEM),
            pl.BlockSpec(memory_space=pltpu.VMEM),
            pl.BlockSpec(memory_space=pltpu.VMEM),
            pl.BlockSpec(memory_space=pltpu.VMEM),
            pl.BlockSpec(memory_space=pltpu.VMEM),
            pl.BlockSpec(memory_space=pltpu.VMEM),
        ],
        out_specs=pl.BlockSpec(memory_space=pltpu.VMEM),
    )(t.reshape(B, 1), speaker_ids.astype(jnp.int32), spk_emb,
      te_w1, te_b1.reshape(1, C), te_w2, te_b2.reshape(1, C),
      spk_w, spk_b.reshape(1, C))


def _inproj_kernel(x_ref, w_ref, b_ref, o_ref):
    o_ref[...] = jnp.dot(x_ref[...], w_ref[...],
                         preferred_element_type=jnp.float32) + b_ref[...]


def _inproj(x_t, in_w, in_b, bn=512):
    return pl.pallas_call(
        _inproj_kernel,
        grid=(N // bn,),
        in_specs=[
            pl.BlockSpec((bn, A), lambda i: (i, 0)),
            pl.BlockSpec((A, D), lambda i: (0, 0)),
            pl.BlockSpec((1, D), lambda i: (0, 0)),
        ],
        out_specs=pl.BlockSpec((bn, D), lambda i: (i, 0)),
        out_shape=jax.ShapeDtypeStruct((N, D), jnp.float32),
    )(x_t.reshape(N, A), in_w, in_b.reshape(1, D))


def _cond_ss_kernel(sem_ref, ts_ref, cw_ref, cb_ref, n1w_ref, n1b_ref,
                    n2w_ref, n2b_ref, ss1_ref, ss2_ref, *, blocks_per_batch):
    bi = pl.program_id(0) // blocks_per_batch
    ts = jnp.where(bi == 0, ts_ref[0:1, :], ts_ref[1:2, :])
    cat = jnp.concatenate(
        [sem_ref[...], jnp.broadcast_to(ts, (sem_ref.shape[0], 2 * C))],
        axis=-1)
    cond = jnp.dot(cat, cw_ref[...],
                   preferred_element_type=jnp.float32) + cb_ref[...]
    ss1_ref[...] = jnp.dot(cond, n1w_ref[...],
                           preferred_element_type=jnp.float32) + n1b_ref[...]
    ss2_ref[...] = jnp.dot(cond, n2w_ref[...],
                           preferred_element_type=jnp.float32) + n2b_ref[...]


def _cond_ss(sem_cond, cond_w, cond_b, ts, n1_w, n1_b, n2_w, n2_b, bn=512):
    blocks_per_batch = T // bn
    return pl.pallas_call(
        functools.partial(_cond_ss_kernel, blocks_per_batch=blocks_per_batch),
        grid=(N // bn,),
        in_specs=[
            pl.BlockSpec((bn, C), lambda i: (i, 0)),
            pl.BlockSpec((B, 2 * C), lambda i: (0, 0)),
            pl.BlockSpec((3 * C, D), lambda i: (0, 0)),
            pl.BlockSpec((1, D), lambda i: (0, 0)),
            pl.BlockSpec((D, 2 * D), lambda i: (0, 0)),
            pl.BlockSpec((1, 2 * D), lambda i: (0, 0)),
            pl.BlockSpec((D, 2 * D), lambda i: (0, 0)),
            pl.BlockSpec((1, 2 * D), lambda i: (0, 0)),
        ],
        out_specs=[
            pl.BlockSpec((bn, 2 * D), lambda i: (i, 0)),
            pl.BlockSpec((bn, 2 * D), lambda i: (i, 0)),
        ],
        out_shape=[
            jax.ShapeDtypeStruct((N, 2 * D), jnp.float32),
            jax.ShapeDtypeStruct((N, 2 * D), jnp.float32),
        ],
    )(sem_cond, ts, cond_w, cond_b.reshape(1, D), n1_w,
      n1_b.reshape(1, 2 * D), n2_w, n2_b.reshape(1, 2 * D))


def _ln_rows(x):
    mu = jnp.mean(x, axis=-1, keepdims=True)
    var = jnp.var(x, axis=-1, keepdims=True)
    return (x - mu) / jnp.sqrt(var + EPS)


def _h1qkv_kernel(x_ref, ss_ref, w_ref, qkv_ref):
    h = _ln_rows(x_ref[...])
    h = h * (1.0 + ss_ref[:, :D]) + ss_ref[:, D:]
    qkv_ref[...] = jnp.dot(h, w_ref[...], preferred_element_type=jnp.float32)


def _h1qkv(x, ss1, qkv_w, bn=512):
    return pl.pallas_call(
        _h1qkv_kernel,
        grid=(N // bn,),
        in_specs=[
            pl.BlockSpec((bn, D), lambda i: (i, 0)),
            pl.BlockSpec((bn, 2 * D), lambda i: (i, 0)),
            pl.BlockSpec((D, 3 * D), lambda i: (0, 0)),
        ],
        out_specs=pl.BlockSpec((bn, 3 * D), lambda i: (i, 0)),
        out_shape=jax.ShapeDtypeStruct((N, 3 * D), jnp.float32),
    )(x, ss1, qkv_w)


def _attn_kernel(q_ref, k_ref, v_ref, o_ref):
    outs = []
    for h in range(2):
        q = q_ref[0][:, h * HD:(h + 1) * HD]
        k = k_ref[0][:, h * HD:(h + 1) * HD]
        v = v_ref[0][:, h * HD:(h + 1) * HD]
        s = lax.dot_general(q, k, (((1,), (1,)), ((), ())),
                            preferred_element_type=jnp.float32)
        s = s / jnp.sqrt(jnp.float32(HD))
        p = jax.nn.softmax(s, axis=-1)
        outs.append(jnp.dot(p, v, preferred_element_type=jnp.float32))
    o_ref[0] = jnp.concatenate(outs, axis=-1)


def _attention(qkv, bq=512):
    hp = H // 2
    return pl.pallas_call(
        _attn_kernel,
        grid=(B, hp, T // bq),
        in_specs=[
            pl.BlockSpec((1, bq, 2 * HD), lambda b, h, i: (b, i, h)),
            pl.BlockSpec((1, T, 2 * HD), lambda b, h, i: (b, 0, hp + h)),
            pl.BlockSpec((1, T, 2 * HD), lambda b, h, i: (b, 0, 2 * hp + h)),
        ],
        out_specs=pl.BlockSpec((1, bq, 2 * HD), lambda b, h, i: (b, i, h)),
        out_shape=jax.ShapeDtypeStruct((B, T, D), jnp.float32),
    )(qkv, qkv, qkv)


def _postattn_kernel(attn_ref, aow_ref, xin_ref, ss_ref, rw_ref,
                     x_ref, h2_ref, comb_ref):
    x = xin_ref[...] + jnp.dot(attn_ref[...], aow_ref[...],
                               preferred_element_type=jnp.float32)
    x_ref[...] = x
    h = _ln_rows(x)
    h2 = h * (1.0 + ss_ref[:, :D]) + ss_ref[:, D:]
    h2_ref[...] = h2
    logits = jnp.dot(h2, rw_ref[...], preferred_element_type=jnp.float32)
    ie = lax.broadcasted_iota(jnp.int32, logits.shape, 1)
    m1 = jnp.max(logits, axis=-1, keepdims=True)
    i1 = jnp.min(jnp.where(logits >= m1, ie, E), axis=-1, keepdims=True)
    l2 = jnp.where(ie == i1, -jnp.inf, logits)
    m2 = jnp.max(l2, axis=-1, keepdims=True)
    i2 = jnp.min(jnp.where(l2 >= m2, ie, E), axis=-1, keepdims=True)
    e2 = jnp.exp(m2 - m1)
    wa = 1.0 / (1.0 + e2)
    wb = e2 * wa
    comb_ref[...] = jnp.where(ie == i1, wa, 0.0) + jnp.where(ie == i2, wb, 0.0)


def _postattn(attn, attn_out_w, x_in, ss2, router_w, bn=512):
    return pl.pallas_call(
        _postattn_kernel,
        grid=(N // bn,),
        in_specs=[
            pl.BlockSpec((bn, D), lambda i: (i, 0)),
            pl.BlockSpec((D, D), lambda i: (0, 0)),
            pl.BlockSpec((bn, D), lambda i: (i, 0)),
            pl.BlockSpec((bn, 2 * D), lambda i: (i, 0)),
            pl.BlockSpec((D, E), lambda i: (0, 0)),
        ],
        out_specs=[
            pl.BlockSpec((bn, D), lambda i: (i, 0)),
            pl.BlockSpec((bn, D), lambda i: (i, 0)),
            pl.BlockSpec((bn, E), lambda i: (i, 0)),
        ],
        out_shape=[
            jax.ShapeDtypeStruct((N, D), jnp.float32),
            jax.ShapeDtypeStruct((N, D), jnp.float32),
            jax.ShapeDtypeStruct((N, E), jnp.float32),
        ],
    )(attn, attn_out_w, x_in, ss2, router_w)


def _moe_dense_kernel(h2_ref, w1_ref, b1_ref, w2_ref, b2_ref, comb_ref, o_ref):
    e = pl.program_id(1)
    f = pl.program_id(2)

    @pl.when((e == 0) & (f == 0))
    def _():
        o_ref[...] = jnp.zeros_like(o_ref)

    h = jnp.dot(h2_ref[...], w1_ref[0], preferred_element_type=jnp.float32)
    h = h + b1_ref[0]
    g = 0.5 * h * (1.0 + lax.erf(h * (2.0 ** -0.5)))
    y = jnp.dot(g, w2_ref[0], preferred_element_type=jnp.float32)
    comb = comb_ref[...]
    ie = lax.broadcasted_iota(jnp.int32, comb.shape, 1)
    c = jnp.sum(jnp.where(ie == e, comb, 0.0), axis=-1, keepdims=True)
    y = y + jnp.where(f == 0, 1.0, 0.0) * b2_ref[0]
    o_ref[...] += c * y


def _moe_dense(h2, comb, e_w1, e_b1, e_w2, e_b2, bn=512, fc=1024):
    nf = FF // fc
    return pl.pallas_call(
        _moe_dense_kernel,
        grid=(N // bn, E, nf),
        in_specs=[
            pl.BlockSpec((bn, D), lambda i, e, f: (i, 0)),
            pl.BlockSpec((1, D, fc), lambda i, e, f: (e, 0, f)),
            pl.BlockSpec((1, 1, fc), lambda i, e, f: (e, 0, f)),
            pl.BlockSpec((1, fc, D), lambda i, e, f: (e, f, 0)),
            pl.BlockSpec((1, 1, D), lambda i, e, f: (e, 0, 0)),
            pl.BlockSpec((bn, E), lambda i, e, f: (i, 0)),
        ],
        out_specs=pl.BlockSpec((bn, D), lambda i, e, f: (i, 0)),
        out_shape=jax.ShapeDtypeStruct((N, D), jnp.float32),
    )(h2, e_w1, e_b1.reshape(E, 1, FF), e_w2, e_b2.reshape(E, 1, D), comb)


def _final_kernel(x_ref, moe_ref, g_ref, b_ref, ow_ref, ob_ref, o_ref):
    x = x_ref[...] + moe_ref[...]
    h = _ln_rows(x) * g_ref[...] + b_ref[...]
    o_ref[...] = jnp.dot(h, ow_ref[...], preferred_element_type=jnp.float32) + ob_ref[...]


def _final(x, x_moe, on_g, on_b, op_w, op_b, bn=512):
    return pl.pallas_call(
        _final_kernel,
        grid=(N // bn,),
        in_specs=[
            pl.BlockSpec((bn, D), lambda i: (i, 0)),
            pl.BlockSpec((bn, D), lambda i: (i, 0)),
            pl.BlockSpec((1, D), lambda i: (0, 0)),
            pl.BlockSpec((1, D), lambda i: (0, 0)),
            pl.BlockSpec((D, A), lambda i: (0, 0)),
            pl.BlockSpec((1, A), lambda i: (0, 0)),
        ],
        out_specs=pl.BlockSpec((bn, A), lambda i: (i, 0)),
        out_shape=jax.ShapeDtypeStruct((N, A), jnp.float32),
    )(x, x_moe, on_g.reshape(1, D), on_b.reshape(1, D), op_w,
      op_b.reshape(1, A))


def kernel(x_t, t, semantic_tokens, speaker_ids, sem_emb, te_w1, te_b1,
           te_w2, te_b2, spk_emb, spk_w, spk_b, cond_w, cond_b, in_w, in_b,
           n1_w, n1_b, qkv_w, attn_out_w, n2_w, n2_b, router_w, e_w1, e_b1,
           e_w2, e_b2, on_g, on_b, op_w, op_b):
    tokens = semantic_tokens.reshape(N).astype(jnp.int32)
    sem_cond = _sem_gather_sc(sem_emb, tokens)
    ts = _prelude(t, speaker_ids, spk_emb, te_w1, te_b1, te_w2, te_b2,
                  spk_w, spk_b)
    x_in = _inproj(x_t, in_w, in_b)
    ss1, ss2 = _cond_ss(sem_cond, cond_w, cond_b, ts, n1_w, n1_b, n2_w, n2_b)
    qkv = _h1qkv(x_in, ss1, qkv_w)
    attn = _attention(qkv.reshape(B, T, 3 * D))
    x, h2, comb = _postattn(attn.reshape(N, D), attn_out_w, x_in, ss2,
                            router_w)
    x_moe = _moe_dense(h2, comb, e_w1, e_b1, e_w2, e_b2)
    out = _final(x, x_moe, on_g, on_b, op_w, op_b)
    return out.reshape(B, T, A)
